# baseline (speedup 1.0000x reference)
import functools

import jax
import jax.numpy as jnp
from jax import lax
from jax.experimental import pallas as pl
from jax.experimental.pallas import tpu as pltpu
from jax.experimental.pallas import tpu_sc as plsc

_B = 16384
_D = 128
_NCORES = 2
_NSUB = 16
_NW = _NCORES * _NSUB
_BPW = _B // _NW

_BM = 2048


_CH = 256


def _sc_gather_body(uid_hbm, qid_hbm, user_t, qdiff_t, qtab_t, qdisc_t,
                    u_out, d_out, m_out, disc_out,
                    uid_v, qid_v, buf0, buf1, disc_v,
                    gsem, ssem0, ssem1, dsem):
    wid = lax.axis_index("s") * _NCORES + lax.axis_index("c")
    base = wid * _BPW
    pltpu.sync_copy(uid_hbm.at[pl.ds(base, _BPW)], uid_v)
    pltpu.sync_copy(qid_hbm.at[pl.ds(base, _BPW)], qid_v)
    gd = pltpu.async_copy(qdisc_t.at[qid_v], disc_v, dsem)
    bufs = (buf0, buf1)
    ssems = (ssem0, ssem1)
    tasks = []
    for tab, out, idx in ((user_t, u_out, uid_v),
                          (qdiff_t, d_out, qid_v),
                          (qtab_t, m_out, qid_v)):
        for c in range(_BPW // _CH):
            tasks.append((tab, out, idx, c * _CH))
    scatters = [None, None]
    gd.wait()
    sd = pltpu.async_copy(disc_v, disc_out.at[pl.ds(base, _BPW)], dsem)
    for k, (tab, out, idx, off) in enumerate(tasks):
        slot = k % 2
        if scatters[slot] is not None:
            scatters[slot].wait()
        g = pltpu.async_copy(tab.at[idx.at[pl.ds(off, _CH)]], bufs[slot], gsem)
        g.wait()
        scatters[slot] = pltpu.async_copy(
            bufs[slot], out.at[pl.ds(base + off, _CH)], ssems[slot])
    scatters[0].wait()
    scatters[1].wait()
    sd.wait()


@functools.cache
def _sc_gather():
    return pl.kernel(
        _sc_gather_body,
        out_type=[
            jax.ShapeDtypeStruct((_B, _D), jnp.float32),
            jax.ShapeDtypeStruct((_B, _D), jnp.float32),
            jax.ShapeDtypeStruct((_B, _D), jnp.float32),
            jax.ShapeDtypeStruct((_B,), jnp.float32),
        ],
        mesh=plsc.VectorSubcoreMesh(core_axis_name="c", subcore_axis_name="s",
                                    num_cores=_NCORES, num_subcores=_NSUB),
        scratch_types=[
            pltpu.VMEM((_BPW,), jnp.int32),
            pltpu.VMEM((_BPW,), jnp.int32),
            pltpu.VMEM((_CH, _D), jnp.float32),
            pltpu.VMEM((_CH, _D), jnp.float32),
            pltpu.VMEM((_BPW,), jnp.float32),
            pltpu.SemaphoreType.DMA,
            pltpu.SemaphoreType.DMA,
            pltpu.SemaphoreType.DMA,
            pltpu.SemaphoreType.DMA,
        ],
    )


def _sigmoid(x):
    return 0.5 * jnp.tanh(0.5 * x) + 0.5


def _mlp_body(u_ref, d_ref, m_ref, disc_ref, w1_ref, b1_ref, w2_ref, b2_ref,
              w3t_ref, b3_ref, out_ref):
    u = _sigmoid(u_ref[...])
    d = _sigmoid(d_ref[...])
    disc = _sigmoid(disc_ref[...]) * 10.0
    x = disc * (u - d) * m_ref[...]
    h = _sigmoid(
        jnp.dot(x, w1_ref[...], preferred_element_type=jnp.float32) + b1_ref[...])
    h = _sigmoid(
        jnp.dot(h, w2_ref[...], preferred_element_type=jnp.float32) + b2_ref[...])
    o = jnp.sum(h * w3t_ref[...], axis=-1, keepdims=True) + b3_ref[...]
    out_ref[...] = _sigmoid(o)


@functools.partial(jax.jit, static_argnames=())
def _ncd_forward(uid, qid, q_table, user_table, q_diff_table, q_disc_table,
                 W1, b1, W2, b2, W3, b3):
    u_rows, d_rows, m_rows, disc = _sc_gather()(
        uid, qid, user_table, q_diff_table, q_table,
        q_disc_table.reshape(-1))
    disc = disc.reshape(_B, 1)

    grid = _B // _BM
    row_spec = pl.BlockSpec((_BM, _D), lambda i: (i, 0))
    col1_spec = pl.BlockSpec((_BM, 1), lambda i: (i, 0))
    full = lambda shape: pl.BlockSpec(shape, lambda i: (0,) * len(shape))
    out = pl.pallas_call(
        _mlp_body,
        grid=(grid,),
        in_specs=[
            row_spec, row_spec, row_spec, col1_spec,
            full((128, 512)), full((1, 512)),
            full((512, 256)), full((1, 256)),
            full((1, 256)), full((1, 1)),
        ],
        out_specs=col1_spec,
        out_shape=jax.ShapeDtypeStruct((_B, 1), jnp.float32),
        compiler_params=pltpu.CompilerParams(
            dimension_semantics=("arbitrary",)),
    )(u_rows, d_rows, m_rows, disc,
      W1, b1.reshape(1, -1), W2, b2.reshape(1, -1),
      W3.reshape(1, -1), b3.reshape(1, 1))
    return out.reshape(-1)


def kernel(user_id, question_id, q_table, user_table, q_diff_table,
           q_disc_table, W1, b1, W2, b2, W3, b3):
    uid = user_id.astype(jnp.int32)
    qid = question_id.astype(jnp.int32)
    return _ncd_forward(uid, qid, q_table, user_table, q_diff_table,
                        q_disc_table, W1, b1, W2, b2, W3, b3)


# speedup vs baseline: 1.1704x; 1.1704x over previous
import functools

import jax
import jax.numpy as jnp
from jax import lax
from jax.experimental import pallas as pl
from jax.experimental.pallas import tpu as pltpu
from jax.experimental.pallas import tpu_sc as plsc

_B = 16384
_D = 128
_NCORES = 2
_NSUB = 16
_NW = _NCORES * _NSUB
_BPW = _B // _NW

_BM = 2048


def _sc_gather_body(uid_hbm, qid_hbm, user_t, qdiff_t, qtab_t, qdisc_t,
                    u_out, d_out, m_out, disc_out,
                    uid_v, qid_v, rows_v, disc_v, sem):
    wid = lax.axis_index("s") * _NCORES + lax.axis_index("c")
    base = wid * _BPW
    pltpu.sync_copy(uid_hbm.at[pl.ds(base, _BPW)], uid_v)
    pltpu.sync_copy(qid_hbm.at[pl.ds(base, _BPW)], qid_v)
    pltpu.async_copy(user_t.at[uid_v], rows_v, sem).wait()
    pltpu.sync_copy(rows_v, u_out.at[pl.ds(base, _BPW)])
    pltpu.async_copy(qdiff_t.at[qid_v], rows_v, sem).wait()
    pltpu.sync_copy(rows_v, d_out.at[pl.ds(base, _BPW)])
    pltpu.async_copy(qtab_t.at[qid_v], rows_v, sem).wait()
    pltpu.sync_copy(rows_v, m_out.at[pl.ds(base, _BPW)])
    pltpu.async_copy(qdisc_t.at[qid_v], disc_v, sem).wait()
    pltpu.sync_copy(disc_v, disc_out.at[pl.ds(base, _BPW)])


@functools.cache
def _sc_gather():
    return pl.kernel(
        _sc_gather_body,
        out_type=[
            jax.ShapeDtypeStruct((_B, _D), jnp.float32),
            jax.ShapeDtypeStruct((_B, _D), jnp.float32),
            jax.ShapeDtypeStruct((_B, _D), jnp.float32),
            jax.ShapeDtypeStruct((_B,), jnp.float32),
        ],
        mesh=plsc.VectorSubcoreMesh(core_axis_name="c", subcore_axis_name="s",
                                    num_cores=_NCORES, num_subcores=_NSUB),
        scratch_types=[
            pltpu.VMEM((_BPW,), jnp.int32),
            pltpu.VMEM((_BPW,), jnp.int32),
            pltpu.VMEM((_BPW, _D), jnp.float32),
            pltpu.VMEM((_BPW,), jnp.float32),
            pltpu.SemaphoreType.DMA,
        ],
    )


def _sigmoid(x):
    return 0.5 * jnp.tanh(0.5 * x) + 0.5


def _mlp_body(u_ref, d_ref, m_ref, disc_ref, w1_ref, b1_ref, w2_ref, b2_ref,
              w3t_ref, b3_ref, out_ref):
    u = _sigmoid(u_ref[...])
    d = _sigmoid(d_ref[...])
    disc = _sigmoid(disc_ref[...].T) * 10.0
    x = disc * (u - d) * m_ref[...]
    h = _sigmoid(
        jnp.dot(x, w1_ref[...], preferred_element_type=jnp.float32) + b1_ref[...])
    h = _sigmoid(
        jnp.dot(h, w2_ref[...], preferred_element_type=jnp.float32) + b2_ref[...])
    o = jnp.sum(h * w3t_ref[...], axis=-1, keepdims=True) + b3_ref[...]
    out_ref[...] = _sigmoid(o).T


@jax.jit
def _ncd_forward(uid, qid, q_table, user_table, q_diff_table, q_disc_table,
                 W1, b1, W2, b2, W3, b3):
    u_rows, d_rows, m_rows, disc = _sc_gather()(
        uid, qid, user_table, q_diff_table, q_table,
        q_disc_table.reshape(-1))

    grid = _B // _BM
    row_spec = pl.BlockSpec((_BM, _D), lambda i: (i, 0))
    rowvec_spec = pl.BlockSpec((1, _BM), lambda i: (0, i))
    full = lambda shape: pl.BlockSpec(shape, lambda i: (0,) * len(shape))
    out = pl.pallas_call(
        _mlp_body,
        grid=(grid,),
        in_specs=[
            row_spec, row_spec, row_spec, rowvec_spec,
            full((128, 512)), full((1, 512)),
            full((512, 256)), full((1, 256)),
            full((1, 256)), full((1, 1)),
        ],
        out_specs=rowvec_spec,
        out_shape=jax.ShapeDtypeStruct((1, _B), jnp.float32),
        compiler_params=pltpu.CompilerParams(
            dimension_semantics=("arbitrary",)),
    )(u_rows, d_rows, m_rows, disc.reshape(1, _B),
      W1, b1.reshape(1, -1), W2, b2.reshape(1, -1),
      W3.reshape(1, -1), b3.reshape(1, 1))
    return out.reshape(-1)


def kernel(user_id, question_id, q_table, user_table, q_diff_table,
           q_disc_table, W1, b1, W2, b2, W3, b3):
    uid = user_id.astype(jnp.int32)
    qid = question_id.astype(jnp.int32)
    return _ncd_forward(uid, qid, q_table, user_table, q_diff_table,
                        q_disc_table, W1, b1, W2, b2, W3, b3)
